# baseline (speedup 1.0000x reference)
import functools

import jax
import jax.numpy as jnp
from jax import lax
from jax.experimental import pallas as pl
from jax.experimental.pallas import tpu as pltpu
from jax.experimental.pallas import tpu_sc as plsc

B, T = 1024, 200
N = B * T
DIMS = (32, 16, 16, 16)
OFFS = (0, 32, 48, 64)
PAD = 128
OUT_DIM = 128

NC, NS = 2, 16
NW = NC * NS
ROWS_PER_W = N // NW
IDX_LANES = 128
IDX_ROWS_PER_W = ROWS_PER_W // IDX_LANES
CHUNK_IDX_ROWS = 5
CHUNK = CHUNK_IDX_ROWS * IDX_LANES
NCHUNK = IDX_ROWS_PER_W // CHUNK_IDX_ROWS

PBN0 = 65536
PBN1 = 8192


def _make_pack(d, pbn, v):
    r = 128 // d
    prows = pbn // r
    nb = -(-v // pbn)

    def body(x, o):
        acc = None
        for g in range(r):
            y = lax.dot_general(
                x[:, prows * g:prows * (g + 1)],
                jnp.eye(d, 128, d * g, dtype=jnp.float32),
                (((0,), (0,)), ((), ())),
                preferred_element_type=jnp.float32)
            acc = y if acc is None else acc + y
        o[...] = acc

    @jax.jit
    def pack(et):
        return pl.pallas_call(
            body,
            grid=(nb,),
            in_specs=[pl.BlockSpec((d, pbn), lambda i: (0, i))],
            out_specs=pl.BlockSpec((prows, PAD), lambda i: (i, 0)),
            out_shape=jax.ShapeDtypeStruct((nb * prows, PAD), jnp.float32),
        )(et)

    def pidx(t):
        return (prows * (t // pbn) + t % prows) * r + (t % pbn) // prows

    return pack, pidx, nb * prows * r


_pack0, _pidx0, P0V = _make_pack(32, PBN0, 1000000)
_pack1, _pidx1, P1V = _make_pack(16, PBN1, 100000)


def _gather_body(f0, f1, f2, f3, e0, e1, e2, e3, out,
                 i0, i1, i2, i3, r0, r1, r2, r3, sem):
    wid = lax.axis_index("s") * NC + lax.axis_index("c")
    base_r = wid * ROWS_PER_W

    pltpu.sync_copy(f0.at[wid], i0)
    pltpu.sync_copy(f1.at[wid], i1)
    pltpu.sync_copy(f2.at[wid], i2)
    pltpu.sync_copy(f3.at[wid], i3)

    tabs = (e0, e1, e2, e3)
    idxs = (i0, i1, i2, i3)
    rbufs = (r0, r1, r2, r3)

    def chunk(c, carry):
        cps = []
        for t in range(4):
            for j in range(CHUNK_IDX_ROWS):
                cps.append(pltpu.make_async_copy(
                    tabs[t].at[idxs[t].at[c * CHUNK_IDX_ROWS + j]],
                    rbufs[t].at[pl.ds(j * IDX_LANES, IDX_LANES)],
                    sem,
                ))
        for cp in cps:
            cp.start()
        for cp in cps:
            cp.wait()
        rows = pl.ds(base_r + c * CHUNK, CHUNK)
        for t in range(4):
            pltpu.sync_copy(rbufs[t], out.at[rows, pl.ds(OFFS[t], DIMS[t])])
        return carry

    lax.fori_loop(0, NCHUNK, chunk, 0)


@jax.jit
def _sc_gather(f0, f1, f2, f3, e0, e1, e2, e3):
    mesh = plsc.VectorSubcoreMesh(core_axis_name="c", subcore_axis_name="s")
    return pl.kernel(
        _gather_body,
        out_type=jax.ShapeDtypeStruct((N, PAD), jnp.float32),
        mesh=mesh,
        scratch_types=[
            pltpu.VMEM((IDX_ROWS_PER_W, IDX_LANES), jnp.int32),
            pltpu.VMEM((IDX_ROWS_PER_W, IDX_LANES), jnp.int32),
            pltpu.VMEM((IDX_ROWS_PER_W, IDX_LANES), jnp.int32),
            pltpu.VMEM((IDX_ROWS_PER_W, IDX_LANES), jnp.int32),
            pltpu.VMEM((CHUNK, 32), jnp.float32),
            pltpu.VMEM((CHUNK, 16), jnp.float32),
            pltpu.VMEM((CHUNK, 16), jnp.float32),
            pltpu.VMEM((CHUNK, 16), jnp.float32),
            pltpu.SemaphoreType.DMA,
        ],
        compiler_params=pltpu.CompilerParams(
            use_tc_tiling_on_sc=False, needs_layout_passes=False),
    )(f0, f1, f2, f3, e0, e1, e2, e3)


MM_BLK = 8192


def _mm_body(s, w, bias, o):
    lane = lax.broadcasted_iota(jnp.int32, (MM_BLK, PAD), 1)
    sv = jnp.where(lane < 80, s[...], 0.0)
    o[...] = jnp.dot(sv, w[...],
                     preferred_element_type=jnp.float32) + bias[0:1, :]


@jax.jit
def _tc_project(s, w, bias):
    return pl.pallas_call(
        _mm_body,
        grid=(N // MM_BLK,),
        in_specs=[
            pl.BlockSpec((MM_BLK, PAD), lambda i: (i, 0)),
            pl.BlockSpec((PAD, OUT_DIM), lambda i: (0, 0)),
            pl.BlockSpec((8, OUT_DIM), lambda i: (0, 0)),
        ],
        out_specs=pl.BlockSpec((MM_BLK, OUT_DIM), lambda i: (i, 0)),
        out_shape=jax.ShapeDtypeStruct((N, OUT_DIM), jnp.float32),
    )(s, w, bias)


def kernel(feat0, feat1, feat2, feat3, E0, E1, E2, E3, W, b):
    shaped = lambda f: f.reshape(NW, IDX_ROWS_PER_W, IDX_LANES)
    g0 = shaped(_pidx0(feat0))
    g1 = shaped(_pidx1(feat1))
    g2 = shaped(_pidx1(feat2))
    g3 = shaped(feat3)
    P0 = _pack0(jnp.transpose(E0)).reshape(P0V, 32)
    P1 = _pack1(jnp.transpose(E1)).reshape(P1V, 16)
    P2 = _pack1(jnp.transpose(E2)).reshape(P1V, 16)
    s = _sc_gather(g0, g1, g2, g3, P0, P1, P2, E3)
    wp = jnp.zeros((PAD, OUT_DIM), jnp.float32).at[0:80, :].set(W)
    bias = jnp.broadcast_to(b, (8, OUT_DIM))
    out = _tc_project(s, wp, bias)
    return out.reshape(B, T, OUT_DIM)


# speedup vs baseline: 1.0110x; 1.0110x over previous
import functools

import jax
import jax.numpy as jnp
from jax import lax
from jax.experimental import pallas as pl
from jax.experimental.pallas import tpu as pltpu
from jax.experimental.pallas import tpu_sc as plsc

B, T = 1024, 200
N = B * T
DIMS = (32, 16, 16, 16)
OFFS = (0, 32, 48, 64)
PAD = 128
OUT_DIM = 128

NC, NS = 2, 16
NW = NC * NS
ROWS_PER_W = N // NW
IDX_LANES = 128
IDX_ROWS_PER_W = ROWS_PER_W // IDX_LANES
CHUNK_IDX_ROWS = 5
CHUNK = CHUNK_IDX_ROWS * IDX_LANES
NCHUNK = IDX_ROWS_PER_W // CHUNK_IDX_ROWS

PBN0 = 32768
PBN1 = 8192


def _make_pack(d, pbn, v):
    r = 128 // d
    prows = pbn // r
    nb = -(-v // pbn)

    def body(x, o):
        acc = None
        for g in range(r):
            y = lax.dot_general(
                x[:, prows * g:prows * (g + 1)],
                jnp.eye(d, 128, d * g, dtype=jnp.float32),
                (((0,), (0,)), ((), ())),
                preferred_element_type=jnp.float32)
            acc = y if acc is None else acc + y
        o[...] = acc

    @jax.jit
    def pack(et):
        return pl.pallas_call(
            body,
            grid=(nb,),
            in_specs=[pl.BlockSpec((d, pbn), lambda i: (0, i))],
            out_specs=pl.BlockSpec((prows, PAD), lambda i: (i, 0)),
            out_shape=jax.ShapeDtypeStruct((nb * prows, PAD), jnp.float32),
        )(et)

    def pidx(t):
        return (prows * (t // pbn) + t % prows) * r + (t % pbn) // prows

    return pack, pidx, nb * prows * r


_pack0, _pidx0, P0V = _make_pack(32, PBN0, 1000000)
_pack1, _pidx1, P1V = _make_pack(16, PBN1, 100000)


def _gather_body(f0, f1, f2, f3, e0, e1, e2, e3, out,
                 i0, i1, i2, i3, r0, r1, r2, r3, sem):
    wid = lax.axis_index("s") * NC + lax.axis_index("c")
    base_r = wid * ROWS_PER_W

    pltpu.sync_copy(f0.at[wid], i0)
    pltpu.sync_copy(f1.at[wid], i1)
    pltpu.sync_copy(f2.at[wid], i2)
    pltpu.sync_copy(f3.at[wid], i3)

    tabs = (e0, e1, e2, e3)
    idxs = (i0, i1, i2, i3)
    rbufs = (r0, r1, r2, r3)

    def chunk(c, carry):
        cps = []
        for t in range(4):
            for j in range(CHUNK_IDX_ROWS):
                cps.append(pltpu.make_async_copy(
                    tabs[t].at[idxs[t].at[c * CHUNK_IDX_ROWS + j]],
                    rbufs[t].at[pl.ds(j * IDX_LANES, IDX_LANES)],
                    sem,
                ))
        for cp in cps:
            cp.start()
        for cp in cps:
            cp.wait()
        rows = pl.ds(base_r + c * CHUNK, CHUNK)
        for t in range(4):
            pltpu.sync_copy(rbufs[t], out.at[rows, pl.ds(OFFS[t], DIMS[t])])
        return carry

    lax.fori_loop(0, NCHUNK, chunk, 0)


@jax.jit
def _sc_gather(f0, f1, f2, f3, e0, e1, e2, e3):
    mesh = plsc.VectorSubcoreMesh(core_axis_name="c", subcore_axis_name="s")
    return pl.kernel(
        _gather_body,
        out_type=jax.ShapeDtypeStruct((N, PAD), jnp.float32),
        mesh=mesh,
        scratch_types=[
            pltpu.VMEM((IDX_ROWS_PER_W, IDX_LANES), jnp.int32),
            pltpu.VMEM((IDX_ROWS_PER_W, IDX_LANES), jnp.int32),
            pltpu.VMEM((IDX_ROWS_PER_W, IDX_LANES), jnp.int32),
            pltpu.VMEM((IDX_ROWS_PER_W, IDX_LANES), jnp.int32),
            pltpu.VMEM((CHUNK, 32), jnp.float32),
            pltpu.VMEM((CHUNK, 16), jnp.float32),
            pltpu.VMEM((CHUNK, 16), jnp.float32),
            pltpu.VMEM((CHUNK, 16), jnp.float32),
            pltpu.SemaphoreType.DMA,
        ],
        compiler_params=pltpu.CompilerParams(
            use_tc_tiling_on_sc=False, needs_layout_passes=False),
    )(f0, f1, f2, f3, e0, e1, e2, e3)


MM_BLK = 8192


def _mm_body(s, w, bias, o):
    lane = lax.broadcasted_iota(jnp.int32, (MM_BLK, PAD), 1)
    sv = jnp.where(lane < 80, s[...], 0.0)
    o[...] = jnp.dot(sv, w[...],
                     preferred_element_type=jnp.float32) + bias[0:1, :]


@jax.jit
def _tc_project(s, w, bias):
    return pl.pallas_call(
        _mm_body,
        grid=(N // MM_BLK,),
        in_specs=[
            pl.BlockSpec((MM_BLK, PAD), lambda i: (i, 0)),
            pl.BlockSpec((PAD, OUT_DIM), lambda i: (0, 0)),
            pl.BlockSpec((8, OUT_DIM), lambda i: (0, 0)),
        ],
        out_specs=pl.BlockSpec((MM_BLK, OUT_DIM), lambda i: (i, 0)),
        out_shape=jax.ShapeDtypeStruct((N, OUT_DIM), jnp.float32),
    )(s, w, bias)


def kernel(feat0, feat1, feat2, feat3, E0, E1, E2, E3, W, b):
    shaped = lambda f: f.reshape(NW, IDX_ROWS_PER_W, IDX_LANES)
    g0 = shaped(_pidx0(feat0))
    g1 = shaped(_pidx1(feat1))
    g2 = shaped(_pidx1(feat2))
    g3 = shaped(feat3)
    P0 = _pack0(jnp.transpose(E0)).reshape(P0V, 32)
    P1 = _pack1(jnp.transpose(E1)).reshape(P1V, 16)
    P2 = _pack1(jnp.transpose(E2)).reshape(P1V, 16)
    s = _sc_gather(g0, g1, g2, g3, P0, P1, P2, E3)
    wp = jnp.zeros((PAD, OUT_DIM), jnp.float32).at[0:80, :].set(W)
    bias = jnp.broadcast_to(b, (8, OUT_DIM))
    out = _tc_project(s, wp, bias)
    return out.reshape(B, T, OUT_DIM)
